# baseline (speedup 1.0000x reference)
import functools

import jax
import jax.numpy as jnp
from jax import lax
from jax.experimental import pallas as pl
from jax.experimental.pallas import tpu as pltpu
from jax.experimental.pallas import tpu_sc as plsc

_B, _C, _H, _W = 8, 2, 512, 512
_PIX = _H * _W
_NPIX = _B * _PIX
_NW = 32
_CHUNK = _NPIX // _NW
_BLK = 4096
_NBLK = _CHUNK // _BLK
_GRPS = _BLK // 64
_VMAX = _BLK // 16
_TAG = 1 << 30
_SENT = 1 << 29
_AMSK = (1 << 23) - 1
_LCAP = _CHUNK + 256
_G = 128


def _softplus(xs, x):
    u = jnp.exp(-jnp.abs(x))
    y = 1.0 + u
    t = u * (6.0 + u) / (6.0 + 4.0 * u)
    t = t - 1.0 + y * jnp.exp(-t)
    t = t - 1.0 + y * jnp.exp(-t)
    return jnp.maximum(xs, 0.0) + t


def _sc_body(pred_ref, lab_ref, up_ref, out_ref,
             lbuf, ubuf, lst, g0, g1, v0, v1, outv,
             sl0, sl1, su0, su1, sg0, sg1):
    cid = lax.axis_index("c")
    sid = lax.axis_index("s")
    wid = cid * 16 + sid
    b = wid // 4
    start = wid * _CHUNK

    iota = lax.iota(jnp.int32, 16)
    perms = [(iota + (1 << s)) % 16 for s in range(4)]
    sentv = jnp.full((16,), _SENT, jnp.int32)
    addr_base = start + b * _PIX

    in_sems = (sl0, sl1)
    up_sems = (su0, su1)
    g_sems = (sg0, sg1)

    def copy_in(g):
        slot = g % 2
        pltpu.async_copy(
            lab_ref.at[pl.ds(start + g * _BLK, _BLK)], lbuf.at[slot], in_sems[slot])
        pltpu.async_copy(
            up_ref.at[pl.ds(start + g * _BLK, _BLK)], ubuf.at[slot], up_sems[slot])

    def wait_in(g):
        slot = g % 2
        pltpu.make_async_copy(
            lab_ref.at[pl.ds(start + g * _BLK, _BLK)], lbuf.at[slot], in_sems[slot]).wait()
        pltpu.make_async_copy(
            up_ref.at[pl.ds(start + g * _BLK, _BLK)], ubuf.at[slot], up_sems[slot]).wait()

    def tree_or(v):
        for p in perms:
            v = v | jnp.take(v, p)
        return v

    def scan_block(g, vcur):
        slot = g % 2
        addr_blk = addr_base + g * _BLK

        def grp(j, vcur):
            ls, us, cms = [], [], []
            for k in range(4):
                off = j * 64 + k * 16
                l = lbuf[slot, pl.ds(off, 16)]
                u = ubuf[slot, pl.ds(off, 16)]
                n_m = l == 255
                nn_m = (l == 0) & (u == 255)
                ls.append(n_m)
                cms.append(n_m | nn_m)
            av = jnp.where(cms[0], 1, 0) | jnp.where(cms[1], 2, 0) \
                | jnp.where(cms[2], 4, 0) | jnp.where(cms[3], 8, 0)
            bits = tree_or(av)[0]

            @pl.when(bits != 0)
            def _():
                pos = vcur
                for k in range(4):
                    bk = (bits >> k) & 1

                    @pl.when(bk != 0)
                    def _(k=k, pos=pos):
                        pix = addr_blk + j * 64 + k * 16 + iota
                        enc = jnp.where(cms[k],
                                        jnp.where(ls[k], pix + _TAG, pix), sentv)
                        lst[pl.ds(pos * 16, 16)] = enc

                    if k < 3:
                        pos = pos + bk

            adv = ((bits & 1) + ((bits >> 1) & 1) + ((bits >> 2) & 1)
                   + ((bits >> 3) & 1))
            return vcur + adv

        return lax.fori_loop(0, _GRPS, grp, vcur)

    def pad_segment(vcur):
        np_ = (8 - (vcur % 8)) % 8

        def padb(i, _):
            lst[pl.ds((vcur + i) * 16, 16)] = sentv
            return 0

        lax.fori_loop(0, np_, padb, 0)
        return vcur + np_

    def fire_gathers(g, pvcur, vcur):
        slot = g % 2
        nch = (vcur - pvcur) // 8

        def fire(i, _):
            for jj in range(8):
                e = lst[pl.ds((pvcur + i * 8 + jj) * 16, 16)]
                a0 = e & _AMSK
                g0[slot, pl.ds(i * _G + jj * 16, 16)] = a0
                g1[slot, pl.ds(i * _G + jj * 16, 16)] = a0 + _PIX
            pltpu.async_copy(
                pred_ref.at[g0.at[slot, pl.ds(i * _G, _G)]],
                v0.at[slot, pl.ds(i * _G, _G)], g_sems[slot])
            pltpu.async_copy(
                pred_ref.at[g1.at[slot, pl.ds(i * _G, _G)]],
                v1.at[slot, pl.ds(i * _G, _G)], g_sems[slot])
            return 0

        lax.fori_loop(0, nch, fire, 0)

    def process_segment(g, pvcur, vcur, carry):
        slot = g % 2
        nch = (vcur - pvcur) // 8

        def proc(i, carry):
            acc, nav, n1v = carry
            pltpu.make_async_copy(
                pred_ref.at[g0.at[slot, pl.ds(i * _G, _G)]],
                v0.at[slot, pl.ds(i * _G, _G)], g_sems[slot]).wait()
            pltpu.make_async_copy(
                pred_ref.at[g1.at[slot, pl.ds(i * _G, _G)]],
                v1.at[slot, pl.ds(i * _G, _G)], g_sems[slot]).wait()
            for jj in range(8):
                e = lst[pl.ds((pvcur + i * 8 + jj) * 16, 16)]
                p0 = v0[slot, pl.ds(i * _G + jj * 16, 16)]
                p1 = v1[slot, pl.ds(i * _G + jj * 16, 16)]
                valid = (e & _SENT) == 0
                tag = (e & _TAG) != 0
                x = p0 - p1
                xs = jnp.where(tag, x, -x)
                val = _softplus(xs, x)
                acc = acc + jnp.where(valid, val, 0.0)
                nav = nav + jnp.where(valid, 1, 0)
                n1v = n1v + jnp.where(tag, 1, 0)
            return (acc, nav, n1v)

        return lax.fori_loop(0, nch, proc, carry)

    copy_in(0)
    carry = (jnp.zeros(16, jnp.float32), jnp.zeros(16, jnp.int32),
             jnp.zeros(16, jnp.int32))
    vcur = jnp.int32(0)
    seg = []
    for g in range(_NBLK):
        wait_in(g)
        if g + 1 < _NBLK:
            copy_in(g + 1)
        pvcur = vcur
        vcur = scan_block(g, vcur)
        vcur = pad_segment(vcur)
        fire_gathers(g, pvcur, vcur)
        seg.append((pvcur, vcur))
        if g > 0:
            carry = process_segment(g - 1, seg[g - 1][0], seg[g - 1][1], carry)
    carry = process_segment(_NBLK - 1, seg[-1][0], seg[-1][1], carry)
    acc, nav, n1v = carry

    def tree_sum(v):
        for p in perms:
            v = v + jnp.take(v, p)
        return v[0]

    s = tree_sum(acc)
    n1 = tree_sum(n1v)
    n = tree_sum(nav)
    triple = jnp.where(iota == 0, s,
                       jnp.where(iota == 1, n1.astype(jnp.float32),
                                 jnp.where(iota == 2, n.astype(jnp.float32), 0.0)))
    outv[...] = triple
    pltpu.sync_copy(outv, out_ref.at[b, pl.ds((wid % 4) * 16, 16)])


@functools.partial(
    pl.kernel,
    mesh=plsc.VectorSubcoreMesh(core_axis_name="c", subcore_axis_name="s"),
    out_type=jax.ShapeDtypeStruct((_B, 64), jnp.float32),
    scratch_types=[
        pltpu.VMEM((2, _BLK), jnp.int32),
        pltpu.VMEM((2, _BLK), jnp.int32),
        pltpu.VMEM((_LCAP,), jnp.int32),
        pltpu.VMEM((2, _VMAX * 16), jnp.int32),
        pltpu.VMEM((2, _VMAX * 16), jnp.int32),
        pltpu.VMEM((2, _VMAX * 16), jnp.float32),
        pltpu.VMEM((2, _VMAX * 16), jnp.float32),
        pltpu.VMEM((16,), jnp.float32),
        pltpu.SemaphoreType.DMA,
        pltpu.SemaphoreType.DMA,
        pltpu.SemaphoreType.DMA,
        pltpu.SemaphoreType.DMA,
        pltpu.SemaphoreType.DMA,
        pltpu.SemaphoreType.DMA,
    ],
)
def _sc_loss(pred_ref, lab_ref, up_ref, out_ref, *scratch):
    _sc_body(pred_ref, lab_ref, up_ref, out_ref, *scratch)


def _combine_kernel(p_ref, o_ref):
    x = p_ref[...]
    s = x[:, 0:1] + x[:, 16:17] + x[:, 32:33] + x[:, 48:49]
    n1 = x[:, 1:2] + x[:, 17:18] + x[:, 33:34] + x[:, 49:50]
    n = x[:, 2:3] + x[:, 18:19] + x[:, 34:35] + x[:, 50:51]
    n2 = n - n1
    ok = (n1 > 0.0) & (n2 > 0.0)
    contrib = jnp.where(ok, s / jnp.where(n > 0.0, n, 1.0), 0.0)
    total = jnp.sum(contrib)
    valid = jnp.sum(ok.astype(jnp.float32))
    o_ref[0] = jnp.where(valid > 0.0, total / jnp.where(valid > 0.0, valid, 1.0), 0.0)


def kernel(predictions, labels, upper_region):
    pred_flat = predictions.reshape(_B * _C * _PIX)
    lab_flat = labels.reshape(_NPIX)
    up_flat = upper_region.reshape(_NPIX)
    partials = _sc_loss(pred_flat, lab_flat, up_flat)
    out = pl.pallas_call(
        _combine_kernel,
        out_specs=pl.BlockSpec(memory_space=pltpu.SMEM),
        out_shape=jax.ShapeDtypeStruct((1,), jnp.float32),
    )(partials)
    return out[0]


# speedup vs baseline: 47.4942x; 47.4942x over previous
import jax
import jax.numpy as jnp
from jax.experimental import pallas as pl
from jax.experimental.pallas import tpu as pltpu

_B, _C, _H, _W = 8, 2, 512, 512
_BPS = 2
_STEPS = _B // _BPS


def _loss_kernel(preds_ref, labels_ref, upper_ref, out_ref, acc_ref):
    g = pl.program_id(0)

    @pl.when(g == 0)
    def _reset_total():
        acc_ref[0] = 0.0
        acc_ref[1] = 0.0

    for i in range(_BPS):
        l = labels_ref[i]
        u = upper_ref[i]
        p0 = preds_ref[i, 0]
        p1 = preds_ref[i, 1]

        neural = l == 255
        nonneural = (l == 0) & (u == 255)
        mask = neural | nonneural

        x = p0 - p1
        sp = jnp.log1p(jnp.exp(-jnp.abs(x)))
        r = jnp.maximum(jnp.where(neural, x, -x), 0.0)
        val = jnp.where(mask, r + sp, 0.0)

        s = jnp.sum(val)
        n1 = jnp.sum(neural.astype(jnp.float32))
        n2 = jnp.sum(nonneural.astype(jnp.float32))

        ok = (n1 > 0.0) & (n2 > 0.0)
        denom = n1 + n2
        contrib = s / jnp.where(denom > 0.0, denom, 1.0)
        acc_ref[0] += jnp.where(ok, contrib, 0.0)
        acc_ref[1] += jnp.where(ok, 1.0, 0.0)

    @pl.when(g == _STEPS - 1)
    def _finish():
        total = acc_ref[0]
        valid = acc_ref[1]
        out_ref[0] = jnp.where(
            valid > 0.0, total / jnp.where(valid > 0.0, valid, 1.0), 0.0
        )


def kernel(predictions, labels, upper_region):
    out = pl.pallas_call(
        _loss_kernel,
        grid=(_STEPS,),
        in_specs=[
            pl.BlockSpec((_BPS, _C, _H, _W), lambda g: (g, 0, 0, 0)),
            pl.BlockSpec((_BPS, _H, _W), lambda g: (g, 0, 0)),
            pl.BlockSpec((_BPS, _H, _W), lambda g: (g, 0, 0)),
        ],
        out_specs=pl.BlockSpec(memory_space=pltpu.SMEM),
        out_shape=jax.ShapeDtypeStruct((1,), jnp.float32),
        scratch_shapes=[pltpu.SMEM((2,), jnp.float32)],
    )(predictions, labels, upper_region)
    return out[0]
